# baseline (speedup 1.0000x reference)
import functools

import jax
import jax.numpy as jnp
from jax import lax
from jax.experimental import pallas as pl
from jax.experimental.pallas import tpu as pltpu
from jax.experimental.pallas import tpu_sc as plsc

_B = 8
_NPTS = 1024
_K = 64
_EPS = 1e-5
_NEG = -1e30


def _fps_body(n, m, px_ref, py_ref, pz_ref, idx_ref, qx_ref, qy_ref, qz_ref):
    px = px_ref[...]
    py = py_ref[...]
    pz = pz_ref[...]
    iota = lax.broadcasted_iota(jnp.int32, (_B, n), 1)
    col = lax.broadcasted_iota(jnp.int32, (_B, m), 1)
    cx = px[:, 0:1]
    cy = py[:, 0:1]
    cz = pz[:, 0:1]
    zf = jnp.zeros((_B, m), jnp.float32)
    acc_qx = jnp.where(col == 0, cx, zf)
    acc_qy = jnp.where(col == 0, cy, zf)
    acc_qz = jnp.where(col == 0, cz, zf)
    acc_ix = jnp.zeros((_B, m), jnp.int32)
    dists = jnp.full((_B, n), jnp.inf, jnp.float32)

    def step(i, carry):
        dists, cx, cy, cz, acc_ix, acc_qx, acc_qy, acc_qz = carry
        dx = px - cx
        dy = py - cy
        dz = pz - cz
        d = dx * dx + dy * dy + dz * dz
        dists = jnp.minimum(dists, d)
        mx = jnp.max(dists, axis=1, keepdims=True)
        cand = jnp.where(dists == mx, iota, n)
        nxt = jnp.min(cand, axis=1, keepdims=True)
        oh = iota == nxt
        ncx = jnp.sum(jnp.where(oh, px, 0.0), axis=1, keepdims=True)
        ncy = jnp.sum(jnp.where(oh, py, 0.0), axis=1, keepdims=True)
        ncz = jnp.sum(jnp.where(oh, pz, 0.0), axis=1, keepdims=True)
        hit = col == i + 1
        acc_ix = jnp.where(hit, nxt, acc_ix)
        acc_qx = jnp.where(hit, ncx, acc_qx)
        acc_qy = jnp.where(hit, ncy, acc_qy)
        acc_qz = jnp.where(hit, ncz, acc_qz)
        return (dists, ncx, ncy, ncz, acc_ix, acc_qx, acc_qy, acc_qz)

    carry = (dists, cx, cy, cz, acc_ix, acc_qx, acc_qy, acc_qz)
    carry = lax.fori_loop(0, m - 1, step, carry)
    idx_ref[...] = carry[4]
    qx_ref[...] = carry[5]
    qy_ref[...] = carry[6]
    qz_ref[...] = carry[7]


def _fps(px, py, pz, m):
    n = px.shape[1]
    outs = (
        jax.ShapeDtypeStruct((_B, m), jnp.int32),
        jax.ShapeDtypeStruct((_B, m), jnp.float32),
        jax.ShapeDtypeStruct((_B, m), jnp.float32),
        jax.ShapeDtypeStruct((_B, m), jnp.float32),
    )
    return pl.pallas_call(
        functools.partial(_fps_body, n, m), out_shape=outs
    )(px, py, pz)


def _sel_body(n, m, rr, qx_ref, qy_ref, qz_ref, si_ref,
              px_ref, py_ref, pz_ref, nbr_ref):
    c = pl.program_id(0)
    qx = qx_ref[0]
    qy = qy_ref[0]
    qz = qz_ref[0]
    sidx = si_ref[0]
    px = px_ref[0]
    py = py_ref[0]
    pz = pz_ref[0]
    dx = qx - px
    dy = qy - py
    dz = qz - pz
    d2 = dx * dx + dy * dy + dz * dz
    iota = lax.broadcasted_iota(jnp.int32, (m, n), 1)
    kio = lax.broadcasted_iota(jnp.int32, (m, _K), 1)
    s0 = jnp.where(d2 <= rr, -d2, _NEG)
    acc0 = jnp.zeros((m, _K), jnp.int32)

    def step(k, carry):
        s, acc = carry
        mx = jnp.max(s, axis=1, keepdims=True)
        valid = mx > (_NEG * 0.5)
        cand = jnp.where(s == mx, iota, n)
        nxt = jnp.min(cand, axis=1, keepdims=True)
        sel = jnp.where(valid, nxt, sidx)
        acc = jnp.where(kio == k, sel + c * n, acc)
        s = jnp.where((iota == nxt) & valid, _NEG, s)
        return (s, acc)

    _, acc = lax.fori_loop(0, _K, step, (s0, acc0))
    nbr_ref[0] = acc


def _select(qx, qy, qz, sidx, px, py, pz, rr):
    m = qx.shape[1]
    n = px.shape[1]
    qspec = pl.BlockSpec((1, m, 1), lambda c: (c, 0, 0))
    pspec = pl.BlockSpec((1, 1, n), lambda c: (c, 0, 0))
    return pl.pallas_call(
        functools.partial(_sel_body, n, m, rr),
        grid=(_B,),
        in_specs=[qspec, qspec, qspec, qspec, pspec, pspec, pspec],
        out_specs=pl.BlockSpec((1, m, _K), lambda c: (c, 0, 0)),
        out_shape=jax.ShapeDtypeStruct((_B, m, _K), jnp.int32),
    )(qx[..., None], qy[..., None], qz[..., None], sidx[..., None],
      px[:, None, :], py[:, None, :], pz[:, None, :])


def _sc_gather(table, idx):
    v, d = table.shape
    bi = idx.shape[0]
    nw = 32
    ch = 128
    per_w = bi // nw
    n_ch = per_w // ch
    mesh = plsc.VectorSubcoreMesh(core_axis_name="c", subcore_axis_name="s")

    @functools.partial(
        pl.kernel,
        mesh=mesh,
        compiler_params=pltpu.CompilerParams(use_tc_tiling_on_sc=False),
        out_type=jax.ShapeDtypeStruct((bi, d), jnp.float32),
        scratch_types=[
            pltpu.VMEM((n_ch, ch), jnp.int32),
            pltpu.VMEM((ch, d), jnp.float32),
            pltpu.VMEM((ch, d), jnp.float32),
            pltpu.SemaphoreType.DMA,
            pltpu.SemaphoreType.DMA,
        ],
    )
    def k(table_hbm, idx_hbm, out_hbm, idx_v, buf0, buf1, sem0, sem1):
        wid = lax.axis_index("s") * 2 + lax.axis_index("c")
        base = wid * per_w
        pltpu.sync_copy(idx_hbm.at[wid], idx_v)

        def step(g, _):
            j0 = 2 * g
            j1 = j0 + 1
            c0 = pltpu.async_copy(table_hbm.at[idx_v.at[j0]], buf0, sem0)
            c1 = pltpu.async_copy(table_hbm.at[idx_v.at[j1]], buf1, sem1)
            c0.wait()
            pltpu.sync_copy(buf0, out_hbm.at[pl.ds(base + j0 * ch, ch)])
            c1.wait()
            pltpu.sync_copy(buf1, out_hbm.at[pl.ds(base + j1 * ch, ch)])
            return 0

        lax.fori_loop(0, n_ch // 2, step, 0)

    return k(table, idx.reshape(nw, n_ch, ch))


def _mlp_body(mb, cout, g_ref, q_ref, w1_ref, b1_ref, w2_ref, b2_ref,
              w3_ref, b3_ref, s1_ref, t1_ref, s2_ref, t2_ref,
              s3_ref, t3_ref, o_ref):
    nq = mb // _K
    g = g_ref[...]
    q = q_ref[...]
    ri = lax.broadcasted_iota(jnp.int32, (mb, nq), 0) // _K
    ci = lax.broadcasted_iota(jnp.int32, (mb, nq), 1)
    e = (ri == ci).astype(jnp.float32)
    u = g - jnp.dot(e, q, preferred_element_type=jnp.float32)
    z1 = jnp.dot(u, w1_ref[...], preferred_element_type=jnp.float32) + b1_ref[...]
    h1 = jnp.maximum(z1, 0.0) * s1_ref[...] + t1_ref[...]
    z2 = jnp.dot(h1, w2_ref[...], preferred_element_type=jnp.float32) + b2_ref[...]
    h2 = jnp.maximum(z2, 0.0) * s2_ref[...] + t2_ref[...]
    z3 = jnp.dot(h2, w3_ref[...], preferred_element_type=jnp.float32) + b3_ref[...]
    h3 = jnp.maximum(z3, 0.0) * s3_ref[...] + t3_ref[...]
    o_ref[...] = jnp.max(h3.reshape(nq, _K, cout), axis=1)


def _conv_mlp(g, q, w1, b1, w2, b2, w3, b3, s1, t1, s2, t2, s3, t3, mb):
    rows, c1 = g.shape
    cout = w3.shape[1]
    grid = rows // mb
    nq = mb // _K

    def wspec(a, b_):
        return pl.BlockSpec((a, b_), lambda i: (0, 0))

    return pl.pallas_call(
        functools.partial(_mlp_body, mb, cout),
        grid=(grid,),
        in_specs=[
            pl.BlockSpec((mb, c1), lambda i: (i, 0)),
            pl.BlockSpec((nq, c1), lambda i: (i, 0)),
            wspec(*w1.shape), wspec(*b1.shape),
            wspec(*w2.shape), wspec(*b2.shape),
            wspec(*w3.shape), wspec(*b3.shape),
            wspec(*s1.shape), wspec(*t1.shape),
            wspec(*s2.shape), wspec(*t2.shape),
            wspec(*s3.shape), wspec(*t3.shape),
        ],
        out_specs=pl.BlockSpec((nq, cout), lambda i: (i, 0)),
        out_shape=jax.ShapeDtypeStruct((rows // _K, cout), jnp.float32),
    )(g, q, w1, b1, w2, b2, w3, b3, s1, t1, s2, t2, s3, t3)


def _final_body(x_ref, p_ref, wa_ref, wp_ref, b1_ref, s1_ref, t1_ref,
                w2_ref, b2_ref, s2_ref, t2_ref,
                w3_ref, b3_ref, s3_ref, t3_ref,
                f1_ref, g1_ref, f2_ref, g2_ref, f3_ref, g3_ref, o_ref):
    x = x_ref[...]
    p = p_ref[...]
    z = (jnp.dot(x, wa_ref[...], preferred_element_type=jnp.float32)
         + jnp.dot(p, wp_ref[...], preferred_element_type=jnp.float32)
         + b1_ref[...])
    h = jnp.maximum(z, 0.0) * s1_ref[...] + t1_ref[...]
    z = jnp.dot(h, w2_ref[...], preferred_element_type=jnp.float32) + b2_ref[...]
    h = jnp.maximum(z, 0.0) * s2_ref[...] + t2_ref[...]
    z = jnp.dot(h, w3_ref[...], preferred_element_type=jnp.float32) + b3_ref[...]
    h = jnp.maximum(z, 0.0) * s3_ref[...] + t3_ref[...]
    gm = jnp.max(h.reshape(_B, 128, 1024), axis=1)
    f = jnp.maximum(
        jnp.dot(gm, f1_ref[...], preferred_element_type=jnp.float32)
        + g1_ref[...], 0.0)
    f = jnp.maximum(
        jnp.dot(f, f2_ref[...], preferred_element_type=jnp.float32)
        + g2_ref[...], 0.0)
    z = jnp.dot(f, f3_ref[...], preferred_element_type=jnp.float32) + g3_ref[...]
    o_ref[...] = 1.0 / (1.0 + jnp.exp(-z))


def _final(x2, p2, args):
    return pl.pallas_call(
        _final_body,
        out_shape=jax.ShapeDtypeStruct((_B, 128), jnp.float32),
    )(x2, p2, *args)


def _bn(lyr):
    s = (lyr["gamma"] / jnp.sqrt(1.0 + _EPS))[None, :]
    t = lyr["beta"][None, :]
    return s, t


def _pad_rows(w, rows):
    return jnp.pad(w, ((0, rows - w.shape[0]), (0, 0)))


def kernel(x, pos, batch, params):
    mlp1, mlp2, mlp3 = params["mlp1"], params["mlp2"], params["mlp3"]
    fc = params["fc"]
    pos_b = pos.reshape(_B, _NPTS, 3)
    px = pos_b[:, :, 0]
    py = pos_b[:, :, 1]
    pz = pos_b[:, :, 2]

    m1 = _NPTS // 2
    idx1, q1x, q1y, q1z = _fps(px, py, pz, m1)
    nbr1 = _select(q1x, q1y, q1z, idx1, px, py, pz, 0.2 * 0.2)

    w1 = mlp1[0]["W"]
    table1 = jnp.pad(jnp.concatenate([x, pos], axis=1), ((0, 0), (0, 11)))
    posq1 = jnp.stack([q1x, q1y, q1z], axis=-1).reshape(_B * m1, 3)
    q1pad = jnp.pad(posq1, ((0, 0), (2, 11)))

    g1 = _sc_gather(table1, nbr1.reshape(-1))
    s1a, t1a = _bn(mlp1[0])
    s1b, t1b = _bn(mlp1[1])
    s1c, t1c = _bn(mlp1[2])
    x1 = _conv_mlp(
        g1, q1pad,
        _pad_rows(w1, 16), mlp1[0]["b"][None, :],
        mlp1[1]["W"], mlp1[1]["b"][None, :],
        mlp1[2]["W"], mlp1[2]["b"][None, :],
        s1a, t1a, s1b, t1b, s1c, t1c, mb=4096)

    m2 = m1 // 4
    idx2, q2x, q2y, q2z = _fps(q1x, q1y, q1z, m2)
    nbr2 = _select(q2x, q2y, q2z, idx2, q1x, q1y, q1z, 0.4 * 0.4)

    w2 = mlp2[0]["W"]
    table2 = jnp.pad(jnp.concatenate([x1, posq1], axis=1), ((0, 0), (0, 13)))
    posq2 = jnp.stack([q2x, q2y, q2z], axis=-1).reshape(_B * m2, 3)
    p2pad = jnp.pad(posq2, ((0, 0), (0, 5)))
    q2pad = jnp.pad(posq2, ((0, 0), (128, 13)))

    g2 = _sc_gather(table2, nbr2.reshape(-1))
    s2a, t2a = _bn(mlp2[0])
    s2b, t2b = _bn(mlp2[1])
    s2c, t2c = _bn(mlp2[2])
    x2 = _conv_mlp(
        g2, q2pad,
        _pad_rows(w2, 144), mlp2[0]["b"][None, :],
        mlp2[1]["W"], mlp2[1]["b"][None, :],
        mlp2[2]["W"], mlp2[2]["b"][None, :],
        s2a, t2a, s2b, t2b, s2c, t2c, mb=4096)

    w3 = mlp3[0]["W"]
    s3a, t3a = _bn(mlp3[0])
    s3b, t3b = _bn(mlp3[1])
    s3c, t3c = _bn(mlp3[2])
    f3w = jnp.pad(fc[2]["W"], ((0, 0), (0, 127)))
    f3b = jnp.pad(fc[2]["b"], (0, 127))[None, :]
    args = (
        w3[:256], _pad_rows(w3[256:259], 8), mlp3[0]["b"][None, :],
        s3a, t3a,
        mlp3[1]["W"], mlp3[1]["b"][None, :], s3b, t3b,
        mlp3[2]["W"], mlp3[2]["b"][None, :], s3c, t3c,
        fc[0]["W"], fc[0]["b"][None, :],
        fc[1]["W"], fc[1]["b"][None, :],
        f3w, f3b,
    )
    out = _final(x2, p2pad, args)
    return out[:, :1]


# speedup vs baseline: 1.1054x; 1.1054x over previous
import functools

import jax
import jax.numpy as jnp
from jax import lax
from jax.experimental import pallas as pl
from jax.experimental.pallas import tpu as pltpu
from jax.experimental.pallas import tpu_sc as plsc

_B = 8
_NPTS = 1024
_K = 64
_EPS = 1e-5
_NEG = -1e30


def _fps_body(n, m, px_ref, py_ref, pz_ref, idx_ref, qx_ref, qy_ref, qz_ref):
    px = px_ref[...]
    py = py_ref[...]
    pz = pz_ref[...]
    iota = lax.broadcasted_iota(jnp.int32, (_B, n), 1)
    col = lax.broadcasted_iota(jnp.int32, (_B, m), 1)
    cx = px[:, 0:1]
    cy = py[:, 0:1]
    cz = pz[:, 0:1]
    zf = jnp.zeros((_B, m), jnp.float32)
    acc_qx = jnp.where(col == 0, cx, zf)
    acc_qy = jnp.where(col == 0, cy, zf)
    acc_qz = jnp.where(col == 0, cz, zf)
    acc_ix = jnp.zeros((_B, m), jnp.int32)
    dists = jnp.full((_B, n), jnp.inf, jnp.float32)

    def step(i, carry):
        dists, cx, cy, cz, acc_ix, acc_qx, acc_qy, acc_qz = carry
        dx = px - cx
        dy = py - cy
        dz = pz - cz
        d = dx * dx + dy * dy + dz * dz
        dists = jnp.minimum(dists, d)
        mx = jnp.max(dists, axis=1, keepdims=True)
        cand = jnp.where(dists == mx, iota, n)
        nxt = jnp.min(cand, axis=1, keepdims=True)
        oh = iota == nxt
        ncx = jnp.sum(jnp.where(oh, px, 0.0), axis=1, keepdims=True)
        ncy = jnp.sum(jnp.where(oh, py, 0.0), axis=1, keepdims=True)
        ncz = jnp.sum(jnp.where(oh, pz, 0.0), axis=1, keepdims=True)
        hit = col == i + 1
        acc_ix = jnp.where(hit, nxt, acc_ix)
        acc_qx = jnp.where(hit, ncx, acc_qx)
        acc_qy = jnp.where(hit, ncy, acc_qy)
        acc_qz = jnp.where(hit, ncz, acc_qz)
        return (dists, ncx, ncy, ncz, acc_ix, acc_qx, acc_qy, acc_qz)

    carry = (dists, cx, cy, cz, acc_ix, acc_qx, acc_qy, acc_qz)
    carry = lax.fori_loop(0, m - 1, step, carry)
    idx_ref[...] = carry[4]
    qx_ref[...] = carry[5]
    qy_ref[...] = carry[6]
    qz_ref[...] = carry[7]


def _fps(px, py, pz, m):
    n = px.shape[1]
    outs = (
        jax.ShapeDtypeStruct((_B, m), jnp.int32),
        jax.ShapeDtypeStruct((_B, m), jnp.float32),
        jax.ShapeDtypeStruct((_B, m), jnp.float32),
        jax.ShapeDtypeStruct((_B, m), jnp.float32),
    )
    return pl.pallas_call(
        functools.partial(_fps_body, n, m), out_shape=outs
    )(px, py, pz)


def _sel_body(n, m, rr, qx_ref, qy_ref, qz_ref, si_ref,
              px_ref, py_ref, pz_ref, nbr_ref):
    c = pl.program_id(0)
    qx = qx_ref[0]
    qy = qy_ref[0]
    qz = qz_ref[0]
    sidx = si_ref[0]
    px = px_ref[0]
    py = py_ref[0]
    pz = pz_ref[0]
    dx = qx - px
    dy = qy - py
    dz = qz - pz
    d2 = dx * dx + dy * dy + dz * dz
    iota = lax.broadcasted_iota(jnp.int32, (m, n), 1)
    kio = lax.broadcasted_iota(jnp.int32, (m, _K), 1)
    s0 = jnp.where(d2 <= rr, -d2, _NEG)
    acc0 = jnp.zeros((m, _K), jnp.int32)
    mx0 = jnp.max(s0, axis=1, keepdims=True)

    def step(k, carry):
        s, mx, acc = carry
        valid = mx > (_NEG * 0.5)
        cand = jnp.where(s == mx, iota, n)
        nxt = jnp.min(cand, axis=1, keepdims=True)
        sel = jnp.where(valid, nxt, sidx)
        acc = jnp.where(kio == k, sel + c * n, acc)
        s = jnp.where(iota == nxt, _NEG, s)
        mx = jnp.max(s, axis=1, keepdims=True)
        return (s, mx, acc)

    _, _, acc = lax.fori_loop(0, _K, step, (s0, mx0, acc0))
    nbr_ref[0] = acc


def _select(qx, qy, qz, sidx, px, py, pz, rr):
    m = qx.shape[1]
    n = px.shape[1]
    qspec = pl.BlockSpec((1, m, 1), lambda c: (c, 0, 0))
    pspec = pl.BlockSpec((1, 1, n), lambda c: (c, 0, 0))
    return pl.pallas_call(
        functools.partial(_sel_body, n, m, rr),
        grid=(_B,),
        in_specs=[qspec, qspec, qspec, qspec, pspec, pspec, pspec],
        out_specs=pl.BlockSpec((1, m, _K), lambda c: (c, 0, 0)),
        out_shape=jax.ShapeDtypeStruct((_B, m, _K), jnp.int32),
    )(qx[..., None], qy[..., None], qz[..., None], sidx[..., None],
      px[:, None, :], py[:, None, :], pz[:, None, :])


def _sc_gather(table, idx):
    v, d = table.shape
    bi = idx.shape[0]
    nw = 32
    ch = 128
    per_w = bi // nw
    n_ch = per_w // ch
    mesh = plsc.VectorSubcoreMesh(core_axis_name="c", subcore_axis_name="s")

    @functools.partial(
        pl.kernel,
        mesh=mesh,
        compiler_params=pltpu.CompilerParams(use_tc_tiling_on_sc=False),
        out_type=jax.ShapeDtypeStruct((bi, d), jnp.float32),
        scratch_types=[
            pltpu.VMEM((n_ch, ch), jnp.int32),
            pltpu.VMEM((ch, d), jnp.float32),
            pltpu.VMEM((ch, d), jnp.float32),
            pltpu.SemaphoreType.DMA,
            pltpu.SemaphoreType.DMA,
        ],
    )
    def k(table_hbm, idx_hbm, out_hbm, idx_v, buf0, buf1, sem0, sem1):
        wid = lax.axis_index("s") * 2 + lax.axis_index("c")
        base = wid * per_w
        pltpu.sync_copy(idx_hbm.at[wid], idx_v)

        def step(g, _):
            j0 = 2 * g
            j1 = j0 + 1
            c0 = pltpu.async_copy(table_hbm.at[idx_v.at[j0]], buf0, sem0)
            c1 = pltpu.async_copy(table_hbm.at[idx_v.at[j1]], buf1, sem1)
            c0.wait()
            pltpu.sync_copy(buf0, out_hbm.at[pl.ds(base + j0 * ch, ch)])
            c1.wait()
            pltpu.sync_copy(buf1, out_hbm.at[pl.ds(base + j1 * ch, ch)])
            return 0

        lax.fori_loop(0, n_ch // 2, step, 0)

    return k(table, idx.reshape(nw, n_ch, ch))


def _mlp_body(mb, cout, g_ref, q_ref, w1_ref, b1_ref, w2_ref, b2_ref,
              w3_ref, b3_ref, s1_ref, t1_ref, s2_ref, t2_ref,
              s3_ref, t3_ref, o_ref):
    nq = mb // _K
    g = g_ref[...]
    q = q_ref[...]
    ri = lax.broadcasted_iota(jnp.int32, (mb, nq), 0) // _K
    ci = lax.broadcasted_iota(jnp.int32, (mb, nq), 1)
    e = (ri == ci).astype(jnp.float32)
    u = g - jnp.dot(e, q, preferred_element_type=jnp.float32)
    z1 = jnp.dot(u, w1_ref[...], preferred_element_type=jnp.float32) + b1_ref[...]
    h1 = jnp.maximum(z1, 0.0) * s1_ref[...] + t1_ref[...]
    z2 = jnp.dot(h1, w2_ref[...], preferred_element_type=jnp.float32) + b2_ref[...]
    h2 = jnp.maximum(z2, 0.0) * s2_ref[...] + t2_ref[...]
    z3 = jnp.dot(h2, w3_ref[...], preferred_element_type=jnp.float32) + b3_ref[...]
    h3 = jnp.maximum(z3, 0.0) * s3_ref[...] + t3_ref[...]
    o_ref[...] = jnp.max(h3.reshape(nq, _K, cout), axis=1)


def _conv_mlp(g, q, w1, b1, w2, b2, w3, b3, s1, t1, s2, t2, s3, t3, mb):
    rows, c1 = g.shape
    cout = w3.shape[1]
    grid = rows // mb
    nq = mb // _K

    def wspec(a, b_):
        return pl.BlockSpec((a, b_), lambda i: (0, 0))

    return pl.pallas_call(
        functools.partial(_mlp_body, mb, cout),
        grid=(grid,),
        in_specs=[
            pl.BlockSpec((mb, c1), lambda i: (i, 0)),
            pl.BlockSpec((nq, c1), lambda i: (i, 0)),
            wspec(*w1.shape), wspec(*b1.shape),
            wspec(*w2.shape), wspec(*b2.shape),
            wspec(*w3.shape), wspec(*b3.shape),
            wspec(*s1.shape), wspec(*t1.shape),
            wspec(*s2.shape), wspec(*t2.shape),
            wspec(*s3.shape), wspec(*t3.shape),
        ],
        out_specs=pl.BlockSpec((nq, cout), lambda i: (i, 0)),
        out_shape=jax.ShapeDtypeStruct((rows // _K, cout), jnp.float32),
    )(g, q, w1, b1, w2, b2, w3, b3, s1, t1, s2, t2, s3, t3)


def _final_body(x_ref, p_ref, wa_ref, wp_ref, b1_ref, s1_ref, t1_ref,
                w2_ref, b2_ref, s2_ref, t2_ref,
                w3_ref, b3_ref, s3_ref, t3_ref,
                f1_ref, g1_ref, f2_ref, g2_ref, f3_ref, g3_ref, o_ref):
    x = x_ref[...]
    p = p_ref[...]
    z = (jnp.dot(x, wa_ref[...], preferred_element_type=jnp.float32)
         + jnp.dot(p, wp_ref[...], preferred_element_type=jnp.float32)
         + b1_ref[...])
    h = jnp.maximum(z, 0.0) * s1_ref[...] + t1_ref[...]
    z = jnp.dot(h, w2_ref[...], preferred_element_type=jnp.float32) + b2_ref[...]
    h = jnp.maximum(z, 0.0) * s2_ref[...] + t2_ref[...]
    z = jnp.dot(h, w3_ref[...], preferred_element_type=jnp.float32) + b3_ref[...]
    h = jnp.maximum(z, 0.0) * s3_ref[...] + t3_ref[...]
    gm = jnp.max(h.reshape(_B, 128, 1024), axis=1)
    f = jnp.maximum(
        jnp.dot(gm, f1_ref[...], preferred_element_type=jnp.float32)
        + g1_ref[...], 0.0)
    f = jnp.maximum(
        jnp.dot(f, f2_ref[...], preferred_element_type=jnp.float32)
        + g2_ref[...], 0.0)
    z = jnp.dot(f, f3_ref[...], preferred_element_type=jnp.float32) + g3_ref[...]
    o_ref[...] = 1.0 / (1.0 + jnp.exp(-z))


def _final(x2, p2, args):
    return pl.pallas_call(
        _final_body,
        out_shape=jax.ShapeDtypeStruct((_B, 128), jnp.float32),
    )(x2, p2, *args)


def _bn(lyr):
    s = (lyr["gamma"] / jnp.sqrt(1.0 + _EPS))[None, :]
    t = lyr["beta"][None, :]
    return s, t


def _pad_rows(w, rows):
    return jnp.pad(w, ((0, rows - w.shape[0]), (0, 0)))


def kernel(x, pos, batch, params):
    mlp1, mlp2, mlp3 = params["mlp1"], params["mlp2"], params["mlp3"]
    fc = params["fc"]
    pos_b = pos.reshape(_B, _NPTS, 3)
    px = pos_b[:, :, 0]
    py = pos_b[:, :, 1]
    pz = pos_b[:, :, 2]

    m1 = _NPTS // 2
    idx1, q1x, q1y, q1z = _fps(px, py, pz, m1)
    nbr1 = _select(q1x, q1y, q1z, idx1, px, py, pz, 0.2 * 0.2)

    w1 = mlp1[0]["W"]
    table1 = jnp.pad(jnp.concatenate([x, pos], axis=1), ((0, 0), (0, 11)))
    posq1 = jnp.stack([q1x, q1y, q1z], axis=-1).reshape(_B * m1, 3)
    q1pad = jnp.pad(posq1, ((0, 0), (2, 11)))

    g1 = _sc_gather(table1, nbr1.reshape(-1))
    s1a, t1a = _bn(mlp1[0])
    s1b, t1b = _bn(mlp1[1])
    s1c, t1c = _bn(mlp1[2])
    x1 = _conv_mlp(
        g1, q1pad,
        _pad_rows(w1, 16), mlp1[0]["b"][None, :],
        mlp1[1]["W"], mlp1[1]["b"][None, :],
        mlp1[2]["W"], mlp1[2]["b"][None, :],
        s1a, t1a, s1b, t1b, s1c, t1c, mb=4096)

    m2 = m1 // 4
    idx2, q2x, q2y, q2z = _fps(q1x, q1y, q1z, m2)
    nbr2 = _select(q2x, q2y, q2z, idx2, q1x, q1y, q1z, 0.4 * 0.4)

    w2 = mlp2[0]["W"]
    table2 = jnp.pad(jnp.concatenate([x1, posq1], axis=1), ((0, 0), (0, 13)))
    posq2 = jnp.stack([q2x, q2y, q2z], axis=-1).reshape(_B * m2, 3)
    p2pad = jnp.pad(posq2, ((0, 0), (0, 5)))
    q2pad = jnp.pad(posq2, ((0, 0), (128, 13)))

    g2 = _sc_gather(table2, nbr2.reshape(-1))
    s2a, t2a = _bn(mlp2[0])
    s2b, t2b = _bn(mlp2[1])
    s2c, t2c = _bn(mlp2[2])
    x2 = _conv_mlp(
        g2, q2pad,
        _pad_rows(w2, 144), mlp2[0]["b"][None, :],
        mlp2[1]["W"], mlp2[1]["b"][None, :],
        mlp2[2]["W"], mlp2[2]["b"][None, :],
        s2a, t2a, s2b, t2b, s2c, t2c, mb=4096)

    w3 = mlp3[0]["W"]
    s3a, t3a = _bn(mlp3[0])
    s3b, t3b = _bn(mlp3[1])
    s3c, t3c = _bn(mlp3[2])
    f3w = jnp.pad(fc[2]["W"], ((0, 0), (0, 127)))
    f3b = jnp.pad(fc[2]["b"], (0, 127))[None, :]
    args = (
        w3[:256], _pad_rows(w3[256:259], 8), mlp3[0]["b"][None, :],
        s3a, t3a,
        mlp3[1]["W"], mlp3[1]["b"][None, :], s3b, t3b,
        mlp3[2]["W"], mlp3[2]["b"][None, :], s3c, t3c,
        fc[0]["W"], fc[0]["b"][None, :],
        fc[1]["W"], fc[1]["b"][None, :],
        f3w, f3b,
    )
    out = _final(x2, p2pad, args)
    return out[:, :1]
